# Initial kernel scaffold; baseline (speedup 1.0000x reference)
import jax
import jax.numpy as jnp
from jax.experimental import pallas as pl


def kernel(edge_index, edge_weight, input_feature, W, b):
    raise NotImplementedError("write your pallas kernel here")


# speedup vs baseline: 4.1107x; 4.1107x over previous
import functools

import jax
import jax.numpy as jnp
from jax import lax
from jax.experimental import pallas as pl
from jax.experimental.pallas import tpu as pltpu
from jax.experimental.pallas import tpu_sc as plsc

_NC = 2
_NS = 16
_L = 16

_BCAST_DNUMS = jax.lax.GatherDimensionNumbers(
    offset_dims=(), collapsed_slice_dims=(0,), start_index_map=(0,))


def _bcast_idx(lane):
    return jnp.full((_L, 1), lane, dtype=jnp.int32)


def _matmul(x, w):
    n, d_in = x.shape
    d_out = w.shape[1]
    bn = 1000

    def mm(x_ref, w_ref, o_ref):
        o_ref[...] = jnp.dot(x_ref[...], w_ref[...],
                             preferred_element_type=jnp.float32)

    return pl.pallas_call(
        mm,
        grid=(n // bn,),
        in_specs=[pl.BlockSpec((bn, d_in), lambda i: (i, 0)),
                  pl.BlockSpec((d_in, d_out), lambda i: (0, 0))],
        out_specs=pl.BlockSpec((bn, d_out), lambda i: (i, 0)),
        out_shape=jax.ShapeDtypeStruct((n, d_out), jnp.float32),
    )(x, w)


def _combine(parts, b):
    _, n, d = parts.shape
    bn = 1000

    def cb(p_ref, b_ref, o_ref):
        o_ref[...] = p_ref[0] + p_ref[1] + b_ref[...]

    return pl.pallas_call(
        cb,
        grid=(n // bn,),
        in_specs=[pl.BlockSpec((2, bn, d), lambda i: (0, i, 0)),
                  pl.BlockSpec((1, d), lambda i: (0, 0))],
        out_specs=pl.BlockSpec((bn, d), lambda i: (i, 0)),
        out_shape=jax.ShapeDtypeStruct((n, d), jnp.float32),
    )(parts, b.reshape(1, d))


def _spmm_partials(support, src, dst, ew):
    n, d = support.shape
    e = src.shape[0]
    nw = _NC * _NS
    epw = e // nw
    chunk = 80
    nchunk = epw // chunk
    rpt = (n // _NS) // 8 * 8
    extra = n - rpt * _NS
    zrows = rpt // 3

    mesh = plsc.VectorSubcoreMesh(core_axis_name="c", subcore_axis_name="s")

    @functools.partial(
        pl.kernel,
        mesh=mesh,
        out_type=jax.ShapeDtypeStruct((_NC, n, d), jnp.float32),
        scratch_types=[
            pltpu.VMEM((chunk,), jnp.int32),
            pltpu.VMEM((chunk,), jnp.int32),
            pltpu.VMEM((chunk,), jnp.float32),
            pltpu.VMEM((chunk, d), jnp.float32),
            pltpu.VMEM((zrows, d), jnp.float32),
            pltpu.VMEM_SHARED((n, d), jnp.float32),
            pltpu.SemaphoreType.DMA,
        ],
    )
    def k(sup_hbm, src_hbm, dst_hbm, ew_hbm, out_hbm,
          src_v, dst_v, ew_v, rows_v, zbuf, acc, sem):
        c = lax.axis_index("c")
        s = lax.axis_index("s")
        wid = s * _NC + c

        zero = jnp.zeros((_L,), jnp.float32)

        def zrow(r, carry):
            for j in range(d // _L):
                zbuf[r, pl.ds(j * _L, _L)] = zero
            return carry

        lax.fori_loop(0, zrows, zrow, 0)
        for t in range(rpt // zrows):
            pltpu.sync_copy(zbuf, acc.at[pl.ds(s * rpt + t * zrows, zrows)])

        @pl.when(s == _NS - 1)
        def _zero_tail():
            pltpu.sync_copy(zbuf.at[pl.ds(0, extra)],
                            acc.at[pl.ds(rpt * _NS, extra)])

        plsc.subcore_barrier()

        ebase = wid * epw

        def do_chunk(kk, carry):
            base = ebase + kk * chunk
            pltpu.sync_copy(src_hbm.at[pl.ds(base, chunk)], src_v)
            pltpu.sync_copy(dst_hbm.at[pl.ds(base, chunk)], dst_v)
            pltpu.sync_copy(ew_hbm.at[pl.ds(base, chunk)], ew_v)
            pltpu.async_copy(sup_hbm.at[src_v], rows_v, sem).wait()

            for g in range(chunk // _L):
                w16 = ew_v[pl.ds(g * _L, _L)]
                for el in range(_L):
                    wbc = lax.gather(
                        w16, _bcast_idx(el), _BCAST_DNUMS, slice_sizes=(1,),
                        mode=lax.GatherScatterMode.PROMISE_IN_BOUNDS)
                    ei = g * _L + el
                    for j in range(d // _L):
                        sl = pl.ds(j * _L, _L)
                        rows_v[ei, sl] = rows_v[ei, sl] * wbc
            pltpu.sync_copy(rows_v, acc.at[dst_v], add=True)
            return carry

        lax.fori_loop(0, nchunk, do_chunk, 0)

        plsc.subcore_barrier()
        pltpu.sync_copy(acc.at[pl.ds(s * rpt, rpt)],
                        out_hbm.at[c, pl.ds(s * rpt, rpt)])

        @pl.when(s == _NS - 1)
        def _write_tail():
            pltpu.sync_copy(acc.at[pl.ds(rpt * _NS, extra)],
                            out_hbm.at[c, pl.ds(rpt * _NS, extra)])

    return k(support, src, dst, ew)


def kernel(edge_index, edge_weight, input_feature, W, b):
    support = _matmul(input_feature, W)
    src = edge_index[0]
    dst = edge_index[1]
    parts = _spmm_partials(support, src, dst, edge_weight)
    return _combine(parts, b)
